# Initial kernel scaffold; baseline (speedup 1.0000x reference)
import jax
import jax.numpy as jnp
from jax.experimental import pallas as pl


def kernel(x, edge_index, edge_attr, batch, params):
    raise NotImplementedError("write your pallas kernel here")


# speedup vs baseline: 2.6166x; 2.6166x over previous
import functools

import jax
import jax.numpy as jnp
from jax import lax
from jax.experimental import pallas as pl
from jax.experimental.pallas import tpu as pltpu
from jax.experimental.pallas import tpu_sc as plsc


def _edge_feat(edge_attr, We, be):
    E, ED = edge_attr.shape
    Dh = We.shape[1]
    EB = 4000
    nb = E // EB

    def body(a_ref, w_ref, b_ref, o_ref):
        o_ref[...] = (
            jnp.dot(a_ref[...], w_ref[...], preferred_element_type=jnp.float32, precision=lax.Precision.HIGHEST)
            + b_ref[...]
        )

    return pl.pallas_call(
        body,
        grid=(nb,),
        in_specs=[
            pl.BlockSpec((EB, ED), lambda i: (i, 0)),
            pl.BlockSpec((ED, Dh), lambda i: (0, 0)),
            pl.BlockSpec((1, Dh), lambda i: (0, 0)),
        ],
        out_specs=pl.BlockSpec((EB, Dh), lambda i: (i, 0)),
        out_shape=jax.ShapeDtypeStruct((E, Dh), jnp.float32),
    )(edge_attr, We, be.reshape(1, Dh))


def _sc_edge_agg(h, e, src, dst):
    Np, Dh = h.shape
    E = src.shape[0]
    NCORE, NSUB = 2, 16
    NW = NCORE * NSUB
    EW = E // NW
    C = 80
    NCHUNK = EW // C
    RPS = (Np // NSUB) // 8 * 8
    RTAIL = RPS - (RPS // C) * C
    REXTRA = Np - NSUB * RPS
    mesh = plsc.VectorSubcoreMesh(core_axis_name="c", subcore_axis_name="s")

    @functools.partial(
        pl.kernel,
        mesh=mesh,
        out_type=jax.ShapeDtypeStruct((NCORE * Np, Dh), jnp.float32),
        scratch_types=[
            pltpu.VMEM((C,), jnp.int32),
            pltpu.VMEM((C,), jnp.int32),
            pltpu.VMEM((C, Dh), jnp.float32),
            pltpu.VMEM((C, Dh), jnp.float32),
            pltpu.VMEM_SHARED((Np, Dh), jnp.float32),
            pltpu.SemaphoreType.DMA,
        ],
    )
    def k(h_hbm, e_hbm, src_hbm, dst_hbm, out_hbm, sidx, didx, ebuf, hbuf, agg, sem):
        c = lax.axis_index("c")
        s = lax.axis_index("s")

        zv = jnp.zeros((16,), jnp.float32)

        def zrow(i, _):
            for j in range(Dh // 16):
                ebuf[i, pl.ds(j * 16, 16)] = zv
            return 0

        lax.fori_loop(0, C, zrow, 0)
        for j in range(RPS // C):
            pltpu.sync_copy(ebuf, agg.at[pl.ds(s * RPS + j * C, C)])
        if RTAIL:
            pltpu.sync_copy(
                ebuf.at[pl.ds(0, RTAIL)],
                agg.at[pl.ds(s * RPS + (RPS // C) * C, RTAIL)],
            )
        if REXTRA:
            @pl.when(s == NSUB - 1)
            def _ztail():
                pltpu.sync_copy(
                    ebuf.at[pl.ds(0, REXTRA)],
                    agg.at[pl.ds(NSUB * RPS, REXTRA)],
                )
        plsc.subcore_barrier()

        base0 = (c * NSUB + s) * EW

        def chunk(j, _):
            b = base0 + j * C
            pltpu.sync_copy(src_hbm.at[pl.ds(b, C)], sidx)
            pltpu.sync_copy(dst_hbm.at[pl.ds(b, C)], didx)
            pltpu.sync_copy(e_hbm.at[pl.ds(b, C)], ebuf)
            pltpu.async_copy(h_hbm.at[sidx], hbuf, sem).wait()

            def row(i, _):
                for jj in range(Dh // 16):
                    sl = pl.ds(jj * 16, 16)
                    ebuf[i, sl] = jnp.maximum(ebuf[i, sl] + hbuf[i, sl], 0.0)
                return 0

            lax.fori_loop(0, C, row, 0)
            pltpu.sync_copy(ebuf, agg.at[didx], add=True)
            return 0

        lax.fori_loop(0, NCHUNK, chunk, 0)
        plsc.subcore_barrier()
        pltpu.sync_copy(
            agg.at[pl.ds(s * RPS, RPS)],
            out_hbm.at[pl.ds(c * Np + s * RPS, RPS)],
        )
        if REXTRA:
            @pl.when(s == NSUB - 1)
            def _wtail():
                pltpu.sync_copy(
                    agg.at[pl.ds(NSUB * RPS, REXTRA)],
                    out_hbm.at[pl.ds(c * Np + NSUB * RPS, REXTRA)],
                )

    return k(h, e, src, dst)


def _mlp(h, agg2, W1, b1, W2, b2):
    Np, Dh = h.shape
    Hh = W1.shape[1]
    NB = 2000
    nb = Np // NB

    def body(h_ref, a0_ref, a1_ref, w1_ref, b1_ref, w2_ref, b2_ref, o_ref):
        z = h_ref[...] + a0_ref[...] + a1_ref[...]
        t = jnp.maximum(
            jnp.dot(z, w1_ref[...], preferred_element_type=jnp.float32, precision=lax.Precision.HIGHEST) + b1_ref[...],
            0.0,
        )
        u = jnp.dot(t, w2_ref[...], preferred_element_type=jnp.float32, precision=lax.Precision.HIGHEST) + b2_ref[...]
        o_ref[...] = jnp.maximum(u, 0.0)

    return pl.pallas_call(
        body,
        grid=(nb,),
        in_specs=[
            pl.BlockSpec((NB, Dh), lambda i: (i, 0)),
            pl.BlockSpec((NB, Dh), lambda i: (i, 0)),
            pl.BlockSpec((NB, Dh), lambda i, _nb=nb: (i + _nb, 0)),
            pl.BlockSpec((Dh, Hh), lambda i: (0, 0)),
            pl.BlockSpec((1, Hh), lambda i: (0, 0)),
            pl.BlockSpec((Hh, Hh), lambda i: (0, 0)),
            pl.BlockSpec((1, Hh), lambda i: (0, 0)),
        ],
        out_specs=pl.BlockSpec((NB, Hh), lambda i: (i, 0)),
        out_shape=jax.ShapeDtypeStruct((Np, Hh), jnp.float32),
    )(h, agg2, agg2, W1, b1.reshape(1, Hh), W2, b2.reshape(1, Hh))


def _pool_head(h, batch3d, Wc1, bc1, Wc2, bc2, G):
    Np, Dh = h.shape
    nb, _, NC = batch3d.shape

    def body(h_ref, b_ref, w1_ref, bb1_ref, w2r_ref, bb2_ref, o_ref, sum_ref, cnt_ref):
        i = pl.program_id(0)

        @pl.when(i == 0)
        def _init():
            sum_ref[...] = jnp.zeros_like(sum_ref)
            cnt_ref[...] = jnp.zeros_like(cnt_ref)

        ids = b_ref[0]
        gi = lax.broadcasted_iota(jnp.int32, (G, NC), 0)
        oh = (ids == gi).astype(jnp.float32)
        sum_ref[...] += jnp.dot(oh, h_ref[...], preferred_element_type=jnp.float32, precision=lax.Precision.HIGHEST)
        cnt_ref[...] += jnp.broadcast_to(jnp.sum(oh, axis=1, keepdims=True), (G, Dh))

        @pl.when(i == nb - 1)
        def _final():
            pooled = sum_ref[...] / jnp.maximum(cnt_ref[...], 1.0)
            hid = jnp.maximum(
                jnp.dot(pooled, w1_ref[...], preferred_element_type=jnp.float32, precision=lax.Precision.HIGHEST)
                + bb1_ref[...],
                0.0,
            )
            res = jnp.sum(hid * w2r_ref[...], axis=1, keepdims=True) + bb2_ref[0, 0]
            o_ref[...] = jnp.broadcast_to(res, (G, Dh))

    out = pl.pallas_call(
        body,
        grid=(nb,),
        in_specs=[
            pl.BlockSpec((NC, Dh), lambda i: (i, 0)),
            pl.BlockSpec((1, 1, NC), lambda i: (i, 0, 0)),
            pl.BlockSpec((Dh, Dh), lambda i: (0, 0)),
            pl.BlockSpec((1, Dh), lambda i: (0, 0)),
            pl.BlockSpec((1, Dh), lambda i: (0, 0)),
            pl.BlockSpec((1, 1), lambda i: (0, 0)),
        ],
        out_specs=pl.BlockSpec((G, Dh), lambda i: (0, 0)),
        out_shape=jax.ShapeDtypeStruct((G, Dh), jnp.float32),
        scratch_shapes=[
            pltpu.VMEM((G, Dh), jnp.float32),
            pltpu.VMEM((G, Dh), jnp.float32),
        ],
    )(h, batch3d, Wc1, bc1.reshape(1, Dh), Wc2.reshape(1, Dh), bc2.reshape(1, 1))
    return out[:, :1]


def kernel(x, edge_index, edge_attr, batch, params):
    src = edge_index[0].astype(jnp.int32)
    dst = edge_index[1].astype(jnp.int32)
    G = 64
    Np = x.shape[0]
    NC = 2000
    batch3d = batch.astype(jnp.int32).reshape(Np // NC, 1, NC)
    h = x
    for lp in params["layers"]:
        e = _edge_feat(edge_attr, lp["We"], lp["be"])
        agg2 = _sc_edge_agg(h, e, src, dst)
        h = _mlp(h, agg2, lp["W1"], lp["b1"], lp["W2"], lp["b2"])
    return _pool_head(h, batch3d, params["Wc1"], params["bc1"], params["Wc2"],
                      params["bc2"], G)


# speedup vs baseline: 4.2420x; 1.6212x over previous
import functools

import jax
import jax.numpy as jnp
from jax import lax
from jax.experimental import pallas as pl
from jax.experimental.pallas import tpu as pltpu
from jax.experimental.pallas import tpu_sc as plsc


def _edge_feat(edge_attr, We, be):
    E, ED = edge_attr.shape
    Dh = We.shape[1]
    EB = 4000
    nb = E // EB

    def body(a_ref, w_ref, b_ref, o_ref):
        o_ref[...] = (
            jnp.dot(a_ref[...], w_ref[...], preferred_element_type=jnp.float32, precision=lax.Precision.HIGHEST)
            + b_ref[...]
        )

    return pl.pallas_call(
        body,
        grid=(nb,),
        in_specs=[
            pl.BlockSpec((EB, ED), lambda i: (i, 0)),
            pl.BlockSpec((ED, Dh), lambda i: (0, 0)),
            pl.BlockSpec((1, Dh), lambda i: (0, 0)),
        ],
        out_specs=pl.BlockSpec((EB, Dh), lambda i: (i, 0)),
        out_shape=jax.ShapeDtypeStruct((E, Dh), jnp.float32),
    )(edge_attr, We, be.reshape(1, Dh))


def _sc_edge_agg(h, e, src, dst):
    Np, Dh = h.shape
    E = src.shape[0]
    NCORE, NSUB = 2, 16
    NW = NCORE * NSUB
    EW = E // NW
    C = 80
    NCHUNK = EW // C
    RPS = (Np // NSUB) // 8 * 8
    RTAIL = RPS - (RPS // C) * C
    REXTRA = Np - NSUB * RPS
    mesh = plsc.VectorSubcoreMesh(core_axis_name="c", subcore_axis_name="s")

    @functools.partial(
        pl.kernel,
        mesh=mesh,
        out_type=jax.ShapeDtypeStruct((NCORE * Np, Dh), jnp.float32),
        scratch_types=[
            pltpu.VMEM((C,), jnp.int32),
            pltpu.VMEM((C,), jnp.int32),
            pltpu.VMEM((C,), jnp.int32),
            pltpu.VMEM((C,), jnp.int32),
            pltpu.VMEM((2, C, Dh), jnp.float32),
            pltpu.VMEM((2, C, Dh), jnp.float32),
            pltpu.VMEM_SHARED((Np, Dh), jnp.float32),
            pltpu.SemaphoreType.DMA,
            pltpu.SemaphoreType.DMA,
            pltpu.SemaphoreType.DMA,
            pltpu.SemaphoreType.DMA,
        ],
    )
    def k(h_hbm, e_hbm, src_hbm, dst_hbm, out_hbm,
          sidx0, sidx1, didx0, didx1, ebuf, hbuf, agg, se0, se1, sh0, sh1):
        c = lax.axis_index("c")
        s = lax.axis_index("s")
        sidx = (sidx0, sidx1)
        didx = (didx0, didx1)
        sems_e = (se0, se1)
        sems_h = (sh0, sh1)

        zv = jnp.zeros((16,), jnp.float32)

        def zrow(i, _):
            for j in range(Dh // 16):
                ebuf[0, i, pl.ds(j * 16, 16)] = zv
            return 0

        lax.fori_loop(0, C, zrow, 0)
        for j in range(RPS // C):
            pltpu.sync_copy(ebuf.at[0], agg.at[pl.ds(s * RPS + j * C, C)])
        if RTAIL:
            pltpu.sync_copy(
                ebuf.at[0, pl.ds(0, RTAIL)],
                agg.at[pl.ds(s * RPS + (RPS // C) * C, RTAIL)],
            )
        if REXTRA:
            @pl.when(s == NSUB - 1)
            def _ztail():
                pltpu.sync_copy(
                    ebuf.at[0, pl.ds(0, REXTRA)],
                    agg.at[pl.ds(NSUB * RPS, REXTRA)],
                )
        plsc.subcore_barrier()

        base0 = (c * NSUB + s) * EW

        def issue(j, b):
            off = base0 + j * C
            pltpu.sync_copy(src_hbm.at[pl.ds(off, C)], sidx[b])
            pltpu.sync_copy(dst_hbm.at[pl.ds(off, C)], didx[b])
            pltpu.async_copy(e_hbm.at[pl.ds(off, C)], ebuf.at[b], sems_e[b])
            pltpu.async_copy(h_hbm.at[sidx[b]], hbuf.at[b], sems_h[b])

        def consume(b):
            pltpu.make_async_copy(e_hbm.at[pl.ds(0, C)], ebuf.at[b], sems_e[b]).wait()
            pltpu.make_async_copy(h_hbm.at[pl.ds(0, C)], hbuf.at[b], sems_h[b]).wait()

            def row(i, _):
                for jj in range(Dh // 16):
                    sl = pl.ds(jj * 16, 16)
                    ebuf[b, i, sl] = jnp.maximum(ebuf[b, i, sl] + hbuf[b, i, sl], 0.0)
                return 0

            lax.fori_loop(0, C, row, 0)
            pltpu.sync_copy(ebuf.at[b], agg.at[didx[b]], add=True)

        issue(0, 0)
        issue(1, 1)

        def pair(g, _):
            j = 2 * g
            for b in range(2):
                consume(b)
                issue(j + b + 2, b)
            return 0

        lax.fori_loop(0, (NCHUNK - 3) // 2, pair, 0)
        consume(0)
        issue(NCHUNK - 1, 0)
        consume(1)
        consume(0)
        plsc.subcore_barrier()
        pltpu.sync_copy(
            agg.at[pl.ds(s * RPS, RPS)],
            out_hbm.at[pl.ds(c * Np + s * RPS, RPS)],
        )
        if REXTRA:
            @pl.when(s == NSUB - 1)
            def _wtail():
                pltpu.sync_copy(
                    agg.at[pl.ds(NSUB * RPS, REXTRA)],
                    out_hbm.at[pl.ds(c * Np + NSUB * RPS, REXTRA)],
                )

    return k(h, e, src, dst)


def _mlp(h, agg2, W1, b1, W2, b2):
    Np, Dh = h.shape
    Hh = W1.shape[1]
    NB = 2000
    nb = Np // NB

    def body(h_ref, a0_ref, a1_ref, w1_ref, b1_ref, w2_ref, b2_ref, o_ref):
        z = h_ref[...] + a0_ref[...] + a1_ref[...]
        t = jnp.maximum(
            jnp.dot(z, w1_ref[...], preferred_element_type=jnp.float32, precision=lax.Precision.HIGHEST) + b1_ref[...],
            0.0,
        )
        u = jnp.dot(t, w2_ref[...], preferred_element_type=jnp.float32, precision=lax.Precision.HIGHEST) + b2_ref[...]
        o_ref[...] = jnp.maximum(u, 0.0)

    return pl.pallas_call(
        body,
        grid=(nb,),
        in_specs=[
            pl.BlockSpec((NB, Dh), lambda i: (i, 0)),
            pl.BlockSpec((NB, Dh), lambda i: (i, 0)),
            pl.BlockSpec((NB, Dh), lambda i, _nb=nb: (i + _nb, 0)),
            pl.BlockSpec((Dh, Hh), lambda i: (0, 0)),
            pl.BlockSpec((1, Hh), lambda i: (0, 0)),
            pl.BlockSpec((Hh, Hh), lambda i: (0, 0)),
            pl.BlockSpec((1, Hh), lambda i: (0, 0)),
        ],
        out_specs=pl.BlockSpec((NB, Hh), lambda i: (i, 0)),
        out_shape=jax.ShapeDtypeStruct((Np, Hh), jnp.float32),
    )(h, agg2, agg2, W1, b1.reshape(1, Hh), W2, b2.reshape(1, Hh))


def _pool_head(h, batch3d, Wc1, bc1, Wc2, bc2, G):
    Np, Dh = h.shape
    nb, _, NC = batch3d.shape

    def body(h_ref, b_ref, w1_ref, bb1_ref, w2r_ref, bb2_ref, o_ref, sum_ref, cnt_ref):
        i = pl.program_id(0)

        @pl.when(i == 0)
        def _init():
            sum_ref[...] = jnp.zeros_like(sum_ref)
            cnt_ref[...] = jnp.zeros_like(cnt_ref)

        ids = b_ref[0]
        gi = lax.broadcasted_iota(jnp.int32, (G, NC), 0)
        oh = (ids == gi).astype(jnp.float32)
        sum_ref[...] += jnp.dot(oh, h_ref[...], preferred_element_type=jnp.float32, precision=lax.Precision.HIGHEST)
        cnt_ref[...] += jnp.broadcast_to(jnp.sum(oh, axis=1, keepdims=True), (G, Dh))

        @pl.when(i == nb - 1)
        def _final():
            pooled = sum_ref[...] / jnp.maximum(cnt_ref[...], 1.0)
            hid = jnp.maximum(
                jnp.dot(pooled, w1_ref[...], preferred_element_type=jnp.float32, precision=lax.Precision.HIGHEST)
                + bb1_ref[...],
                0.0,
            )
            res = jnp.sum(hid * w2r_ref[...], axis=1, keepdims=True) + bb2_ref[0, 0]
            o_ref[...] = jnp.broadcast_to(res, (G, Dh))

    out = pl.pallas_call(
        body,
        grid=(nb,),
        in_specs=[
            pl.BlockSpec((NC, Dh), lambda i: (i, 0)),
            pl.BlockSpec((1, 1, NC), lambda i: (i, 0, 0)),
            pl.BlockSpec((Dh, Dh), lambda i: (0, 0)),
            pl.BlockSpec((1, Dh), lambda i: (0, 0)),
            pl.BlockSpec((1, Dh), lambda i: (0, 0)),
            pl.BlockSpec((1, 1), lambda i: (0, 0)),
        ],
        out_specs=pl.BlockSpec((G, Dh), lambda i: (0, 0)),
        out_shape=jax.ShapeDtypeStruct((G, Dh), jnp.float32),
        scratch_shapes=[
            pltpu.VMEM((G, Dh), jnp.float32),
            pltpu.VMEM((G, Dh), jnp.float32),
        ],
    )(h, batch3d, Wc1, bc1.reshape(1, Dh), Wc2.reshape(1, Dh), bc2.reshape(1, 1))
    return out[:, :1]


def kernel(x, edge_index, edge_attr, batch, params):
    src = edge_index[0].astype(jnp.int32)
    dst = edge_index[1].astype(jnp.int32)
    G = 64
    Np = x.shape[0]
    NC = 2000
    batch3d = batch.astype(jnp.int32).reshape(Np // NC, 1, NC)
    h = x
    for lp in params["layers"]:
        e = _edge_feat(edge_attr, lp["We"], lp["be"])
        agg2 = _sc_edge_agg(h, e, src, dst)
        h = _mlp(h, agg2, lp["W1"], lp["b1"], lp["W2"], lp["b2"])
    return _pool_head(h, batch3d, params["Wc1"], params["bc1"], params["Wc2"],
                      params["bc2"], G)
